# baseline (speedup 1.0000x reference)
import functools

import jax
import jax.numpy as jnp
from jax import lax
from jax.experimental import pallas as pl
from jax.experimental.pallas import tpu as pltpu
from jax.experimental.pallas import tpu_sc as plsc

N = 10000
N_PAD = 10240
E = 320000
D = 128
DW = 16

NC = 2
NS = 16
NW = NC * NS
CHUNK = 128
CPW = 80
NP_AGG = 2
CPS = CPW // NP_AGG
E_PAD = NW * CPW * CHUNK
ROWS_PER_SUB = N_PAD // NS

BR = 1280
GRID = N_PAD // BR

_sc_mesh = plsc.VectorSubcoreMesh(core_axis_name="c", subcore_axis_name="s")


E_PW = CPW * CHUNK


def _sc_deg_body(dst3_hbm, ones_hbm, zeros_hbm, out_hbm, dst_v, ones_v, acc_sh):
    c = lax.axis_index("c")
    s = lax.axis_index("s")
    wid = s * NC + c
    pltpu.sync_copy(dst3_hbm.at[wid], dst_v)
    pltpu.sync_copy(ones_hbm, ones_v)
    sl = pl.ds(s * ROWS_PER_SUB, ROWS_PER_SUB)
    pltpu.sync_copy(zeros_hbm.at[sl], acc_sh.at[sl])
    plsc.subcore_barrier()

    def body(i, carry):
        pltpu.sync_copy(ones_v, acc_sh.at[dst_v.at[i]], add=True)
        return carry

    lax.fori_loop(0, CPW, body, 0)
    plsc.subcore_barrier()
    pltpu.sync_copy(acc_sh.at[sl],
                    out_hbm.at[pl.ds(c * N_PAD + s * ROWS_PER_SUB, ROWS_PER_SUB)])


def _sc_agg_body(src3_hbm, dst3_hbm, g_hbm, zeros_hbm, out_hbm,
                 src_v, dst_v, rows0_v, rows1_v, acc_sh, sem0, sem1):
    c = lax.axis_index("c")
    s = lax.axis_index("s")
    wid = s * NC + c
    sl = pl.ds(s * ROWS_PER_SUB, ROWS_PER_SUB)
    pltpu.sync_copy(zeros_hbm.at[sl], acc_sh.at[sl])
    plsc.subcore_barrier()

    for p in range(NP_AGG):
        base = wid * NP_AGG + p
        pltpu.sync_copy(src3_hbm.at[base], src_v)
        pltpu.sync_copy(dst3_hbm.at[base], dst_v)

        def body(j, carry):
            pltpu.async_copy(g_hbm.at[src_v.at[j]], rows0_v, sem0).wait()
            pltpu.sync_copy(rows0_v, acc_sh.at[dst_v.at[j]], add=True)
            return carry

        lax.fori_loop(0, CPS, body, 0)

    plsc.subcore_barrier()
    pltpu.sync_copy(acc_sh.at[sl], out_hbm.at[pl.ds(c * N_PAD + s * ROWS_PER_SUB, ROWS_PER_SUB)])


_sc_deg = functools.partial(
    pl.kernel,
    out_type=jax.ShapeDtypeStruct((2 * N_PAD, D), jnp.float32),
    mesh=_sc_mesh,
    scratch_types=[
        pltpu.VMEM((CPW, CHUNK), jnp.int32),
        pltpu.VMEM((CHUNK, D), jnp.float32),
        pltpu.VMEM_SHARED((N_PAD, D), jnp.float32),
    ],
)(_sc_deg_body)

_sc_agg = functools.partial(
    pl.kernel,
    out_type=jax.ShapeDtypeStruct((2 * N_PAD, D), jnp.float32),
    mesh=_sc_mesh,
    scratch_types=[
        pltpu.VMEM((CPS, CHUNK), jnp.int32),
        pltpu.VMEM((CPS, CHUNK), jnp.int32),
        pltpu.VMEM((CHUNK, D), jnp.float32),
        pltpu.VMEM((CHUNK, D), jnp.float32),
        pltpu.VMEM_SHARED((N_PAD, D), jnp.float32),
        pltpu.SemaphoreType.DMA,
        pltpu.SemaphoreType.DMA,
    ],
)(_sc_agg_body)


def _dinv_block(degp_blk):
    deg = degp_blk[0, :, 0:1] + degp_blk[1, :, 0:1] + 1.0
    return lax.rsqrt(deg)


def _tc_b_body(degp, x, w1, g1_ref):
    dinv = _dinv_block(degp)
    g1_ref[...] = dinv * jnp.dot(x[...], w1[...], preferred_element_type=jnp.float32)


def _tc_d_body(degp, aggp, g1, b1, w2, g2_ref):
    dinv = _dinv_block(degp)
    t = jnp.maximum(dinv * (aggp[0] + aggp[1] + g1[...]) + b1[...], 0.0)
    g2_ref[...] = dinv * jnp.dot(t, w2[...], preferred_element_type=jnp.float32)


def _tc_e_body(degp, aggp, g2, b2, wm, bm, basis, cw,
               z_ref, cos_ref, out_ref, acc):
    i = pl.program_id(0)
    dinv = _dinv_block(degp)
    h2 = jnp.maximum(dinv * (aggp[0] + aggp[1] + g2[...]) + b2[...], 0.0)
    z = jnp.dot(h2, wm[...], preferred_element_type=jnp.float32) + bm[...]
    z_ref[...] = z
    cs = jnp.dot(z, basis[...], preferred_element_type=jnp.float32)
    nrm = jnp.sqrt(jnp.sum(z * z, axis=1, keepdims=True)) + 1e-8
    cos = cs / nrm
    cos_ref[...] = cos
    rows = i * BR + lax.broadcasted_iota(jnp.int32, (BR, 1), 0)
    psum = jnp.sum(jnp.where(rows < N, cos, 0.0), axis=0, keepdims=True)

    @pl.when(i == 0)
    def _():
        acc[...] = jnp.zeros_like(acc)
        out_ref[...] = jnp.zeros_like(out_ref)

    acc[...] += psum

    @pl.when(i == GRID - 1)
    def _():
        out_ref[...] = jnp.dot(acc[...] * (1.0 / N), cw[...],
                               preferred_element_type=jnp.float32)


_tc_b = pl.pallas_call(
    _tc_b_body,
    grid=(GRID,),
    in_specs=[
        pl.BlockSpec((2, BR, D), lambda i: (0, i, 0)),
        pl.BlockSpec((BR, D), lambda i: (i, 0)),
        pl.BlockSpec((D, D), lambda i: (0, 0)),
    ],
    out_specs=pl.BlockSpec((BR, D), lambda i: (i, 0)),
    out_shape=jax.ShapeDtypeStruct((N_PAD, D), jnp.float32),
)

_tc_d = pl.pallas_call(
    _tc_d_body,
    grid=(GRID,),
    in_specs=[
        pl.BlockSpec((2, BR, D), lambda i: (0, i, 0)),
        pl.BlockSpec((2, BR, D), lambda i: (0, i, 0)),
        pl.BlockSpec((BR, D), lambda i: (i, 0)),
        pl.BlockSpec((1, D), lambda i: (0, 0)),
        pl.BlockSpec((D, D), lambda i: (0, 0)),
    ],
    out_specs=pl.BlockSpec((BR, D), lambda i: (i, 0)),
    out_shape=jax.ShapeDtypeStruct((N_PAD, D), jnp.float32),
)

_OUT_COLS = 40
_NCLS = 10

_tc_e = pl.pallas_call(
    _tc_e_body,
    grid=(GRID,),
    in_specs=[
        pl.BlockSpec((2, BR, D), lambda i: (0, i, 0)),
        pl.BlockSpec((2, BR, D), lambda i: (0, i, 0)),
        pl.BlockSpec((BR, D), lambda i: (i, 0)),
        pl.BlockSpec((1, D), lambda i: (0, 0)),
        pl.BlockSpec((D, D), lambda i: (0, 0)),
        pl.BlockSpec((1, D), lambda i: (0, 0)),
        pl.BlockSpec((D, _OUT_COLS), lambda i: (0, 0)),
        pl.BlockSpec((_OUT_COLS, _NCLS), lambda i: (0, 0)),
    ],
    out_specs=[
        pl.BlockSpec((BR, D), lambda i: (i, 0)),
        pl.BlockSpec((BR, _OUT_COLS), lambda i: (i, 0)),
        pl.BlockSpec((1, _NCLS), lambda i: (0, 0)),
    ],
    out_shape=[
        jax.ShapeDtypeStruct((N_PAD, D), jnp.float32),
        jax.ShapeDtypeStruct((N_PAD, _OUT_COLS), jnp.float32),
        jax.ShapeDtypeStruct((1, _NCLS), jnp.float32),
    ],
    scratch_shapes=[pltpu.VMEM((1, _OUT_COLS), jnp.float32)],
)


def kernel(x, edge_index, W1, b1, W2, b2, Wm, bm, basis_concepts, classifier_weights):
    src = edge_index[0]
    dst = edge_index[1]
    pad = jnp.full((E_PAD - E,), N, jnp.int32)
    srcp = jnp.concatenate([src, pad])
    dstp = jnp.concatenate([dst, pad])
    src3 = srcp.reshape(NW * NP_AGG, CPS, CHUNK)
    dst3 = dstp.reshape(NW * NP_AGG, CPS, CHUNK)
    dst3_deg = dstp.reshape(NW, CPW, CHUNK)
    xp = jnp.pad(x, ((0, N_PAD - N), (0, 0)))
    zeros_d = jnp.zeros((N_PAD, D), jnp.float32)
    ones_d = jnp.ones((CHUNK, D), jnp.float32)
    b1r = b1.reshape(1, D)
    b2r = b2.reshape(1, D)
    bmr = bm.reshape(1, D)

    degp = _sc_deg(dst3_deg, ones_d, zeros_d).reshape(2, N_PAD, D)
    g1 = _tc_b(degp, xp, W1)
    agg1 = _sc_agg(src3, dst3, g1, zeros_d).reshape(2, N_PAD, D)
    g2 = _tc_d(degp, agg1, g1, b1r, W2)
    agg2 = _sc_agg(src3, dst3, g2, zeros_d).reshape(2, N_PAD, D)
    z, cos, out = _tc_e(degp, agg2, g2, b2r, Wm, bmr,
                        basis_concepts, classifier_weights)
    return out, z[:N], cos[:N]


# speedup vs baseline: 1.4545x; 1.4545x over previous
import functools

import jax
import jax.numpy as jnp
from jax import lax
from jax.experimental import pallas as pl
from jax.experimental.pallas import tpu as pltpu
from jax.experimental.pallas import tpu_sc as plsc

N = 10000
N_PAD = 10240
E = 320000
D = 128
DW = 16

NC = 2
NS = 16
NW = NC * NS
CHUNK = 128
CPW = 79
E_PAD = NW * CPW * CHUNK
ROWS_PER_SUB = N_PAD // NS

BR = 1280
GRID = N_PAD // BR

_sc_mesh = plsc.VectorSubcoreMesh(core_axis_name="c", subcore_axis_name="s")


E_PW = CPW * CHUNK


def _sc_deg_body(dst3_hbm, ones_hbm, zeros_hbm, out_hbm, dst_v, ones_v, acc_sh):
    c = lax.axis_index("c")
    s = lax.axis_index("s")
    wid = s * NC + c
    pltpu.sync_copy(dst3_hbm.at[wid], dst_v)
    pltpu.sync_copy(ones_hbm, ones_v)
    sl = pl.ds(s * ROWS_PER_SUB, ROWS_PER_SUB)
    pltpu.sync_copy(zeros_hbm.at[sl], acc_sh.at[sl])
    plsc.subcore_barrier()

    def body(i, carry):
        pltpu.sync_copy(ones_v, acc_sh.at[dst_v.at[i]], add=True)
        return carry

    lax.fori_loop(0, CPW, body, 0)
    plsc.subcore_barrier()
    pltpu.sync_copy(acc_sh.at[sl],
                    out_hbm.at[pl.ds(c * N_PAD + s * ROWS_PER_SUB, ROWS_PER_SUB)])


def _sc_agg_body(src3_hbm, dst3_hbm, g_hbm, zeros_hbm, out_hbm,
                 src_v, dst_v, rows_v, acc_sh, sem):
    c = lax.axis_index("c")
    s = lax.axis_index("s")
    wid = s * NC + c
    pltpu.sync_copy(src3_hbm.at[wid], src_v)
    pltpu.sync_copy(dst3_hbm.at[wid], dst_v)
    sl = pl.ds(s * ROWS_PER_SUB, ROWS_PER_SUB)
    pltpu.sync_copy(zeros_hbm.at[sl], acc_sh.at[sl])
    plsc.subcore_barrier()

    def body(i, carry):
        pltpu.async_copy(g_hbm.at[src_v.at[i]], rows_v, sem).wait()
        pltpu.sync_copy(rows_v, acc_sh.at[dst_v.at[i]], add=True)
        return carry

    lax.fori_loop(0, CPW, body, 0)
    plsc.subcore_barrier()
    pltpu.sync_copy(acc_sh.at[sl], out_hbm.at[pl.ds(c * N_PAD + s * ROWS_PER_SUB, ROWS_PER_SUB)])


_sc_deg = functools.partial(
    pl.kernel,
    out_type=jax.ShapeDtypeStruct((2 * N_PAD, D), jnp.float32),
    mesh=_sc_mesh,
    scratch_types=[
        pltpu.VMEM((CPW, CHUNK), jnp.int32),
        pltpu.VMEM((CHUNK, D), jnp.float32),
        pltpu.VMEM_SHARED((N_PAD, D), jnp.float32),
    ],
)(_sc_deg_body)

_sc_agg = functools.partial(
    pl.kernel,
    out_type=jax.ShapeDtypeStruct((2 * N_PAD, D), jnp.float32),
    mesh=_sc_mesh,
    scratch_types=[
        pltpu.VMEM((CPW, CHUNK), jnp.int32),
        pltpu.VMEM((CPW, CHUNK), jnp.int32),
        pltpu.VMEM((CHUNK, D), jnp.float32),
        pltpu.VMEM_SHARED((N_PAD, D), jnp.float32),
        pltpu.SemaphoreType.DMA,
    ],
)(_sc_agg_body)


def _dinv_block(degp_blk):
    deg = degp_blk[0, :, 0:1] + degp_blk[1, :, 0:1] + 1.0
    return lax.rsqrt(deg)


def _tc_b_body(degp, x, w1, g1_ref):
    dinv = _dinv_block(degp)
    g1_ref[...] = dinv * jnp.dot(x[...], w1[...], preferred_element_type=jnp.float32)


def _tc_d_body(degp, aggp, g1, b1, w2, g2_ref):
    dinv = _dinv_block(degp)
    t = jnp.maximum(dinv * (aggp[0] + aggp[1] + g1[...]) + b1[...], 0.0)
    g2_ref[...] = dinv * jnp.dot(t, w2[...], preferred_element_type=jnp.float32)


def _tc_e_body(degp, aggp, g2, b2, wm, bm, basis, cw,
               z_ref, cos_ref, out_ref, acc):
    i = pl.program_id(0)
    dinv = _dinv_block(degp)
    h2 = jnp.maximum(dinv * (aggp[0] + aggp[1] + g2[...]) + b2[...], 0.0)
    z = jnp.dot(h2, wm[...], preferred_element_type=jnp.float32) + bm[...]
    z_ref[...] = z
    cs = jnp.dot(z, basis[...], preferred_element_type=jnp.float32)
    nrm = jnp.sqrt(jnp.sum(z * z, axis=1, keepdims=True)) + 1e-8
    cos = cs / nrm
    cos_ref[...] = cos
    rows = i * BR + lax.broadcasted_iota(jnp.int32, (BR, 1), 0)
    psum = jnp.sum(jnp.where(rows < N, cos, 0.0), axis=0, keepdims=True)

    @pl.when(i == 0)
    def _():
        acc[...] = jnp.zeros_like(acc)
        out_ref[...] = jnp.zeros_like(out_ref)

    acc[...] += psum

    @pl.when(i == GRID - 1)
    def _():
        out_ref[...] = jnp.dot(acc[...] * (1.0 / N), cw[...],
                               preferred_element_type=jnp.float32)


_tc_b = pl.pallas_call(
    _tc_b_body,
    grid=(GRID,),
    in_specs=[
        pl.BlockSpec((2, BR, D), lambda i: (0, i, 0)),
        pl.BlockSpec((BR, D), lambda i: (i, 0)),
        pl.BlockSpec((D, D), lambda i: (0, 0)),
    ],
    out_specs=pl.BlockSpec((BR, D), lambda i: (i, 0)),
    out_shape=jax.ShapeDtypeStruct((N_PAD, D), jnp.float32),
)

_tc_d = pl.pallas_call(
    _tc_d_body,
    grid=(GRID,),
    in_specs=[
        pl.BlockSpec((2, BR, D), lambda i: (0, i, 0)),
        pl.BlockSpec((2, BR, D), lambda i: (0, i, 0)),
        pl.BlockSpec((BR, D), lambda i: (i, 0)),
        pl.BlockSpec((1, D), lambda i: (0, 0)),
        pl.BlockSpec((D, D), lambda i: (0, 0)),
    ],
    out_specs=pl.BlockSpec((BR, D), lambda i: (i, 0)),
    out_shape=jax.ShapeDtypeStruct((N_PAD, D), jnp.float32),
)

_OUT_COLS = 40
_NCLS = 10

_tc_e = pl.pallas_call(
    _tc_e_body,
    grid=(GRID,),
    in_specs=[
        pl.BlockSpec((2, BR, D), lambda i: (0, i, 0)),
        pl.BlockSpec((2, BR, D), lambda i: (0, i, 0)),
        pl.BlockSpec((BR, D), lambda i: (i, 0)),
        pl.BlockSpec((1, D), lambda i: (0, 0)),
        pl.BlockSpec((D, D), lambda i: (0, 0)),
        pl.BlockSpec((1, D), lambda i: (0, 0)),
        pl.BlockSpec((D, _OUT_COLS), lambda i: (0, 0)),
        pl.BlockSpec((_OUT_COLS, _NCLS), lambda i: (0, 0)),
    ],
    out_specs=[
        pl.BlockSpec((BR, D), lambda i: (i, 0)),
        pl.BlockSpec((BR, _OUT_COLS), lambda i: (i, 0)),
        pl.BlockSpec((1, _NCLS), lambda i: (0, 0)),
    ],
    out_shape=[
        jax.ShapeDtypeStruct((N_PAD, D), jnp.float32),
        jax.ShapeDtypeStruct((N_PAD, _OUT_COLS), jnp.float32),
        jax.ShapeDtypeStruct((1, _NCLS), jnp.float32),
    ],
    scratch_shapes=[pltpu.VMEM((1, _OUT_COLS), jnp.float32)],
)


def kernel(x, edge_index, W1, b1, W2, b2, Wm, bm, basis_concepts, classifier_weights):
    src = edge_index[0]
    dst = edge_index[1]
    pad = jnp.full((E_PAD - E,), N, jnp.int32)
    srcp = jnp.concatenate([src, pad])
    dstp = jnp.concatenate([dst, pad])
    src3 = srcp.reshape(NW, CPW, CHUNK)
    dst3 = dstp.reshape(NW, CPW, CHUNK)
    dst3_deg = dstp.reshape(NW, CPW, CHUNK)
    xp = jnp.pad(x, ((0, N_PAD - N), (0, 0)))
    zeros_d = jnp.zeros((N_PAD, D), jnp.float32)
    ones_d = jnp.ones((CHUNK, D), jnp.float32)
    b1r = b1.reshape(1, D)
    b2r = b2.reshape(1, D)
    bmr = bm.reshape(1, D)

    degp = _sc_deg(dst3_deg, ones_d, zeros_d).reshape(2, N_PAD, D)
    g1 = _tc_b(degp, xp, W1)
    agg1 = _sc_agg(src3, dst3, g1, zeros_d).reshape(2, N_PAD, D)
    g2 = _tc_d(degp, agg1, g1, b1r, W2)
    agg2 = _sc_agg(src3, dst3, g2, zeros_d).reshape(2, N_PAD, D)
    z, cos, out = _tc_e(degp, agg2, g2, b2r, Wm, bmr,
                        basis_concepts, classifier_weights)
    return out, z[:N], cos[:N]


# speedup vs baseline: 1.9714x; 1.3554x over previous
import functools

import jax
import jax.numpy as jnp
from jax import lax
from jax.experimental import pallas as pl
from jax.experimental.pallas import tpu as pltpu
from jax.experimental.pallas import tpu_sc as plsc

N = 10000
N_PAD = 10240
E = 320000
D = 128
DW = 16

NC = 2
NS = 16
NW = NC * NS
CHUNK = 128
CPW = 79
E_PAD = NW * CPW * CHUNK
CPW0 = 98
CPW1 = 59
E_PAD_AGG = NS * (CPW0 + CPW1) * CHUNK
ROWS_PER_SUB = N_PAD // NS

BR = 1280
GRID = N_PAD // BR

_sc_mesh = plsc.VectorSubcoreMesh(core_axis_name="c", subcore_axis_name="s")


E_PW = CPW * CHUNK


def _sc_deg_body(dst3_hbm, ones_hbm, zeros_hbm, out_hbm, dst_v, ones_v, acc_sh):
    c = lax.axis_index("c")
    s = lax.axis_index("s")
    wid = s * NC + c
    pltpu.sync_copy(dst3_hbm.at[wid], dst_v)
    pltpu.sync_copy(ones_hbm, ones_v)
    sl = pl.ds(s * ROWS_PER_SUB, ROWS_PER_SUB)
    pltpu.sync_copy(zeros_hbm.at[sl], acc_sh.at[sl])
    plsc.subcore_barrier()

    def body(i, carry):
        pltpu.sync_copy(ones_v, acc_sh.at[dst_v.at[i]], add=True)
        return carry

    lax.fori_loop(0, CPW, body, 0)
    plsc.subcore_barrier()
    pltpu.sync_copy(acc_sh.at[sl],
                    out_hbm.at[pl.ds(c * N_PAD + s * ROWS_PER_SUB, ROWS_PER_SUB)])


def _sc_agg_body(src3_hbm, dst3_hbm, g_hbm, zeros_hbm, out_hbm,
                 src_v, dst_v, rows_v, acc_sh, sem):
    c = lax.axis_index("c")
    s = lax.axis_index("s")
    wid = s * NC + c
    pltpu.sync_copy(src3_hbm.at[wid], src_v)
    pltpu.sync_copy(dst3_hbm.at[wid], dst_v)
    sl = pl.ds(s * ROWS_PER_SUB, ROWS_PER_SUB)
    pltpu.sync_copy(zeros_hbm.at[sl], acc_sh.at[sl])
    plsc.subcore_barrier()

    def body(i, carry):
        pltpu.async_copy(g_hbm.at[src_v.at[i]], rows_v, sem).wait()
        pltpu.sync_copy(rows_v, acc_sh.at[dst_v.at[i]], add=True)
        return carry

    n_chunks = lax.select(c == 0, jnp.int32(CPW0), jnp.int32(CPW1))
    lax.fori_loop(0, n_chunks, body, 0)
    plsc.subcore_barrier()
    pltpu.sync_copy(acc_sh.at[sl], out_hbm.at[pl.ds(c * N_PAD + s * ROWS_PER_SUB, ROWS_PER_SUB)])


_sc_deg = functools.partial(
    pl.kernel,
    out_type=jax.ShapeDtypeStruct((2 * N_PAD, D), jnp.float32),
    mesh=_sc_mesh,
    scratch_types=[
        pltpu.VMEM((CPW, CHUNK), jnp.int32),
        pltpu.VMEM((CHUNK, D), jnp.float32),
        pltpu.VMEM_SHARED((N_PAD, D), jnp.float32),
    ],
)(_sc_deg_body)

_sc_agg = functools.partial(
    pl.kernel,
    out_type=jax.ShapeDtypeStruct((2 * N_PAD, D), jnp.float32),
    mesh=_sc_mesh,
    scratch_types=[
        pltpu.VMEM((CPW0, CHUNK), jnp.int32),
        pltpu.VMEM((CPW0, CHUNK), jnp.int32),
        pltpu.VMEM((CHUNK, D), jnp.float32),
        pltpu.VMEM_SHARED((N_PAD, D), jnp.float32),
        pltpu.SemaphoreType.DMA,
    ],
)(_sc_agg_body)


def _dinv_block(degp_blk):
    deg = degp_blk[0, :, 0:1] + degp_blk[1, :, 0:1] + 1.0
    return lax.rsqrt(deg)


def _tc_b_body(degp, x, w1, g1_ref):
    dinv = _dinv_block(degp)
    g1_ref[...] = dinv * jnp.dot(x[...], w1[...], preferred_element_type=jnp.float32)


def _tc_d_body(degp, aggp, g1, b1, w2, g2_ref):
    dinv = _dinv_block(degp)
    t = jnp.maximum(dinv * (aggp[0] + aggp[1] + g1[...]) + b1[...], 0.0)
    g2_ref[...] = dinv * jnp.dot(t, w2[...], preferred_element_type=jnp.float32)


def _tc_e_body(degp, aggp, g2, b2, wm, bm, basis, cw,
               z_ref, cos_ref, out_ref, acc):
    i = pl.program_id(0)
    dinv = _dinv_block(degp)
    h2 = jnp.maximum(dinv * (aggp[0] + aggp[1] + g2[...]) + b2[...], 0.0)
    z = jnp.dot(h2, wm[...], preferred_element_type=jnp.float32) + bm[...]
    z_ref[...] = z
    cs = jnp.dot(z, basis[...], preferred_element_type=jnp.float32)
    nrm = jnp.sqrt(jnp.sum(z * z, axis=1, keepdims=True)) + 1e-8
    cos = cs / nrm
    cos_ref[...] = cos
    rows = i * BR + lax.broadcasted_iota(jnp.int32, (BR, 1), 0)
    psum = jnp.sum(jnp.where(rows < N, cos, 0.0), axis=0, keepdims=True)

    @pl.when(i == 0)
    def _():
        acc[...] = jnp.zeros_like(acc)
        out_ref[...] = jnp.zeros_like(out_ref)

    acc[...] += psum

    @pl.when(i == GRID - 1)
    def _():
        out_ref[...] = jnp.dot(acc[...] * (1.0 / N), cw[...],
                               preferred_element_type=jnp.float32)


_tc_b = pl.pallas_call(
    _tc_b_body,
    grid=(GRID,),
    in_specs=[
        pl.BlockSpec((2, BR, D), lambda i: (0, i, 0)),
        pl.BlockSpec((BR, D), lambda i: (i, 0)),
        pl.BlockSpec((D, D), lambda i: (0, 0)),
    ],
    out_specs=pl.BlockSpec((BR, D), lambda i: (i, 0)),
    out_shape=jax.ShapeDtypeStruct((N_PAD, D), jnp.float32),
)

_tc_d = pl.pallas_call(
    _tc_d_body,
    grid=(GRID,),
    in_specs=[
        pl.BlockSpec((2, BR, D), lambda i: (0, i, 0)),
        pl.BlockSpec((2, BR, D), lambda i: (0, i, 0)),
        pl.BlockSpec((BR, D), lambda i: (i, 0)),
        pl.BlockSpec((1, D), lambda i: (0, 0)),
        pl.BlockSpec((D, D), lambda i: (0, 0)),
    ],
    out_specs=pl.BlockSpec((BR, D), lambda i: (i, 0)),
    out_shape=jax.ShapeDtypeStruct((N_PAD, D), jnp.float32),
)

_OUT_COLS = 40
_NCLS = 10

_tc_e = pl.pallas_call(
    _tc_e_body,
    grid=(GRID,),
    in_specs=[
        pl.BlockSpec((2, BR, D), lambda i: (0, i, 0)),
        pl.BlockSpec((2, BR, D), lambda i: (0, i, 0)),
        pl.BlockSpec((BR, D), lambda i: (i, 0)),
        pl.BlockSpec((1, D), lambda i: (0, 0)),
        pl.BlockSpec((D, D), lambda i: (0, 0)),
        pl.BlockSpec((1, D), lambda i: (0, 0)),
        pl.BlockSpec((D, _OUT_COLS), lambda i: (0, 0)),
        pl.BlockSpec((_OUT_COLS, _NCLS), lambda i: (0, 0)),
    ],
    out_specs=[
        pl.BlockSpec((BR, D), lambda i: (i, 0)),
        pl.BlockSpec((BR, _OUT_COLS), lambda i: (i, 0)),
        pl.BlockSpec((1, _NCLS), lambda i: (0, 0)),
    ],
    out_shape=[
        jax.ShapeDtypeStruct((N_PAD, D), jnp.float32),
        jax.ShapeDtypeStruct((N_PAD, _OUT_COLS), jnp.float32),
        jax.ShapeDtypeStruct((1, _NCLS), jnp.float32),
    ],
    scratch_shapes=[pltpu.VMEM((1, _OUT_COLS), jnp.float32)],
)


def kernel(x, edge_index, W1, b1, W2, b2, Wm, bm, basis_concepts, classifier_weights):
    src = edge_index[0]
    dst = edge_index[1]
    def agg_layout(idx):
        idxp = jnp.concatenate([idx, jnp.full((E_PAD_AGG - E,), N, jnp.int32)])
        n0 = NS * CPW0 * CHUNK
        e0 = idxp[:n0].reshape(NS, 1, CPW0, CHUNK)
        e1 = idxp[n0:].reshape(NS, 1, CPW1, CHUNK)
        e1 = jnp.pad(e1, ((0, 0), (0, 0), (0, CPW0 - CPW1), (0, 0)),
                     constant_values=N)
        return jnp.concatenate([e0, e1], axis=1).reshape(NW, CPW0, CHUNK)

    src3 = agg_layout(src)
    dst3 = agg_layout(dst)
    dstp = jnp.concatenate([dst, jnp.full((E_PAD - E,), N, jnp.int32)])
    dst3_deg = dstp.reshape(NW, CPW, CHUNK)
    xp = jnp.pad(x, ((0, N_PAD - N), (0, 0)))
    zeros_d = jnp.zeros((N_PAD, D), jnp.float32)
    ones_d = jnp.ones((CHUNK, D), jnp.float32)
    b1r = b1.reshape(1, D)
    b2r = b2.reshape(1, D)
    bmr = bm.reshape(1, D)

    degp = _sc_deg(dst3_deg, ones_d, zeros_d).reshape(2, N_PAD, D)
    g1 = _tc_b(degp, xp, W1)
    agg1 = _sc_agg(src3, dst3, g1, zeros_d).reshape(2, N_PAD, D)
    g2 = _tc_d(degp, agg1, g1, b1r, W2)
    agg2 = _sc_agg(src3, dst3, g2, zeros_d).reshape(2, N_PAD, D)
    z, cos, out = _tc_e(degp, agg2, g2, b2r, Wm, bmr,
                        basis_concepts, classifier_weights)
    return out, z[:N], cos[:N]
